# baseline (speedup 1.0000x reference)
import math

import jax
import jax.numpy as jnp
import numpy as np
from jax.experimental import pallas as pl
from jax.experimental.pallas import tpu as pltpu

_INV_SQRT2 = 1.0 / math.sqrt(2.0)
_VMEM_LIMIT = 64 * 1024 * 1024


def _fpn1_body(x_ref, w1a_ref, b1a_ref, w1b_ref, b1b_ref, o_ref):
    x = x_ref[0]
    w1b = w1b_ref[...]
    b1b = b1b_ref[...]
    for p in range(4):
        z = jnp.dot(w1a_ref[p], x, preferred_element_type=jnp.float32) + b1a_ref[p]
        z = 0.5 * z * (1.0 + jax.lax.erf(z * _INV_SQRT2))
        y = jnp.dot(w1b, z, preferred_element_type=jnp.float32) + b1b
        o_ref[0, p] = y


def _deconv_body(x_ref, w_ref, b_ref, o_ref):
    o_ref[0] = jnp.dot(w_ref[...], x_ref[0], preferred_element_type=jnp.float32) + b_ref[...]


def _maxpool_body(x_ref, sel_ref, o_ref):
    v = x_ref[...]
    c0 = jnp.dot(v, sel_ref[0], preferred_element_type=jnp.float32)
    c1 = jnp.dot(v, sel_ref[1], preferred_element_type=jnp.float32)
    c2 = jnp.dot(v, sel_ref[2], preferred_element_type=jnp.float32)
    c3 = jnp.dot(v, sel_ref[3], preferred_element_type=jnp.float32)
    o_ref[...] = jnp.maximum(jnp.maximum(c0, c1), jnp.maximum(c2, c3))


def kernel(f0, f1, f2, f3, k_w1a, k_b1a, k_w1b, k_b1b, k_w2, k_b2):
    N, C0, H, W = f0.shape
    cmid = k_w1a.shape[1]
    cout1 = k_w1b.shape[0] // 4
    cout2 = k_w2.shape[0] // 4
    HW = H * W
    t = 512

    x0 = f0.reshape(N, C0, HW)
    o1 = pl.pallas_call(
        _fpn1_body,
        out_shape=jax.ShapeDtypeStruct((N, 4, 4 * cout1, HW), jnp.float32),
        grid=(N, HW // t),
        in_specs=[
            pl.BlockSpec((1, C0, t), lambda n, j: (n, 0, j)),
            pl.BlockSpec((4, cmid, C0), lambda n, j: (0, 0, 0)),
            pl.BlockSpec((4, cmid, 1), lambda n, j: (0, 0, 0)),
            pl.BlockSpec((4 * cout1, cmid), lambda n, j: (0, 0)),
            pl.BlockSpec((4 * cout1, 1), lambda n, j: (0, 0)),
        ],
        out_specs=pl.BlockSpec((1, 4, 4 * cout1, t), lambda n, j: (n, 0, 0, j)),
        compiler_params=pltpu.CompilerParams(
            dimension_semantics=("parallel", "parallel"),
            vmem_limit_bytes=_VMEM_LIMIT,
        ),
    )(x0, k_w1a, k_b1a, k_w1b, k_b1b)
    y1 = o1.reshape(N, 2, 2, cout1, 2, 2, H, W)
    y1 = y1.transpose(0, 3, 6, 1, 4, 7, 2, 5).reshape(N, cout1, 4 * H, 4 * W)

    x1 = f1.reshape(N, C0, HW)
    o2 = pl.pallas_call(
        _deconv_body,
        out_shape=jax.ShapeDtypeStruct((N, 4 * cout2, HW), jnp.float32),
        grid=(N, HW // t),
        in_specs=[
            pl.BlockSpec((1, C0, t), lambda n, j: (n, 0, j)),
            pl.BlockSpec((4 * cout2, C0), lambda n, j: (0, 0)),
            pl.BlockSpec((4 * cout2, 1), lambda n, j: (0, 0)),
        ],
        out_specs=pl.BlockSpec((1, 4 * cout2, t), lambda n, j: (n, 0, j)),
        compiler_params=pltpu.CompilerParams(
            dimension_semantics=("parallel", "parallel"),
            vmem_limit_bytes=_VMEM_LIMIT,
        ),
    )(x1, k_w2, k_b2)
    y2 = o2.reshape(N, cout2, 2, 2, H, W)
    y2 = y2.transpose(0, 1, 4, 2, 5, 3).reshape(N, cout2, 2 * H, 2 * W)

    Nc, C, Hp, Wp = f3.shape
    Hh, Wh = Hp // 2, Wp // 2
    R = Nc * C * Hh
    x3 = f3.reshape(R, 2 * Wp)
    tr = 1024
    i = np.arange(2 * Wp)[None, :, None]
    j = np.arange(Wh)[None, None, :]
    k = np.arange(4)[:, None, None]
    src = (k // 2) * Wp + 2 * j + (k % 2)
    sel = jnp.asarray((i == src).astype(np.float32))
    o4 = pl.pallas_call(
        _maxpool_body,
        out_shape=jax.ShapeDtypeStruct((R, Wh), f3.dtype),
        grid=(R // tr,),
        in_specs=[
            pl.BlockSpec((tr, 2 * Wp), lambda r: (r, 0)),
            pl.BlockSpec((4, 2 * Wp, Wh), lambda r: (0, 0, 0)),
        ],
        out_specs=pl.BlockSpec((tr, Wh), lambda r: (r, 0)),
        compiler_params=pltpu.CompilerParams(
            dimension_semantics=("parallel",),
            vmem_limit_bytes=_VMEM_LIMIT,
        ),
    )(x3, sel)
    y4 = o4.reshape(Nc, C, Hh, Wh)

    return [y1, y2, f2, y4]


# speedup vs baseline: 1.5088x; 1.5088x over previous
import math

import jax
import jax.numpy as jnp
import numpy as np
from jax.experimental import pallas as pl
from jax.experimental.pallas import tpu as pltpu

_INV_SQRT2 = 1.0 / math.sqrt(2.0)
_VMEM_LIMIT = 64 * 1024 * 1024


def _fpn1_body(x_ref, w1a_ref, b1a_ref, w1b_ref, b1b_ref, g_ref, o_ref):
    cout = o_ref.shape[1]
    xb = x_ref[0].astype(jnp.bfloat16)
    z = jnp.dot(w1a_ref[...], xb, preferred_element_type=jnp.float32) + b1a_ref[...]
    z = 0.5 * z * (1.0 + jax.lax.erf(z * _INV_SQRT2))
    zb = z.astype(jnp.bfloat16)
    cmid = zb.shape[0] // 4
    z_cat = jnp.concatenate([zb[p * cmid:(p + 1) * cmid] for p in range(4)], axis=1)
    y = jnp.dot(w1b_ref[...], z_cat, preferred_element_type=jnp.float32) + b1b_ref[...]
    yb = y.astype(jnp.bfloat16)
    rows = []
    for a in range(4):
        kh1, kh2 = a // 2, a % 2
        blocks = []
        for cp in range(4):
            kw1, kw2 = cp // 2, cp % 2
            p = 2 * kh1 + kw1
            r = 2 * kh2 + kw2
            blocks.append(yb[r * cout:(r + 1) * cout, p * 128:(p + 1) * 128])
        rows.append(jnp.concatenate(blocks, axis=1))
    b_in = jnp.concatenate(rows, axis=0)
    b_st = jnp.dot(b_in, g_ref[...], preferred_element_type=jnp.float32)
    for a in range(4):
        for h in range(4):
            o_ref[0, :, 4 * h + a, :] = b_st[a * cout:(a + 1) * cout, 128 * h:128 * (h + 1)]


def _fpn2_body(x_ref, w_ref, b_ref, g_ref, o_ref):
    cout = o_ref.shape[1]
    xb = x_ref[0].astype(jnp.bfloat16)
    y = jnp.dot(w_ref[...], xb, preferred_element_type=jnp.float32) + b_ref[...]
    yb = y.astype(jnp.bfloat16)
    b_in = jnp.concatenate([yb[k * cout:(k + 1) * cout] for k in range(4)], axis=1)
    b_st = jnp.dot(b_in, g_ref[...], preferred_element_type=jnp.float32)
    for h in range(4):
        o_ref[0, :, 0, h, :] = b_st[:, 128 * h:128 * (h + 1)]


def _maxpool_body(x_ref, sel_ref, o_ref):
    v = x_ref[...]
    c0 = jnp.dot(v, sel_ref[0], preferred_element_type=jnp.float32)
    c1 = jnp.dot(v, sel_ref[1], preferred_element_type=jnp.float32)
    c2 = jnp.dot(v, sel_ref[2], preferred_element_type=jnp.float32)
    c3 = jnp.dot(v, sel_ref[3], preferred_element_type=jnp.float32)
    o_ref[...] = jnp.maximum(jnp.maximum(c0, c1), jnp.maximum(c2, c3))


def _stretch_mat(W, maps):
    n = len(maps)
    g = np.zeros((n * 4 * W, 4 * 4 * W), np.float32)
    for b, fn in enumerate(maps):
        for h in range(4):
            for w in range(W):
                g[b * 4 * W + 32 * h + w, fn(h, w)] = 1.0
    return g


def kernel(f0, f1, f2, f3, k_w1a, k_b1a, k_w1b, k_b1b, k_w2, k_b2):
    N, C0, H, W = f0.shape
    cmid = k_w1a.shape[1]
    cout1 = k_w1b.shape[0] // 4
    cout2 = k_w2.shape[0] // 4
    HW = H * W
    bf = jnp.bfloat16

    w1a_cat = k_w1a.reshape(4 * cmid, C0).astype(bf)
    b1a_cat = k_b1a.reshape(4 * cmid, 1)
    w1b_r = k_w1b.reshape(cout1, 4, cmid).transpose(1, 0, 2).reshape(4 * cout1, cmid).astype(bf)
    b1b_r = k_b1b.reshape(cout1, 4).transpose(1, 0).reshape(4 * cout1, 1)
    w2_r = k_w2.reshape(cout2, 4, C0).transpose(1, 0, 2).reshape(4 * cout2, C0).astype(bf)
    b2_r = k_b2.reshape(cout2, 4).transpose(1, 0).reshape(4 * cout2, 1)

    g1 = jnp.asarray(_stretch_mat(W, [
        (lambda cp: (lambda h, w: 128 * h + 4 * w + cp))(cp) for cp in range(4)
    ])).astype(bf)
    g2 = jnp.asarray(_stretch_mat(W, [
        (lambda k: (lambda h, w: 128 * h + 64 * (k // 2) + 2 * w + (k % 2)))(k)
        for k in range(4)
    ])).astype(bf)

    x0 = f0.reshape(N, C0, HW)
    nq = H // 4
    y1 = pl.pallas_call(
        _fpn1_body,
        out_shape=jax.ShapeDtypeStruct((N, cout1, 4 * H, 4 * W), jnp.float32),
        grid=(N, nq),
        in_specs=[
            pl.BlockSpec((1, C0, 4 * W), lambda n, j: (n, 0, j)),
            pl.BlockSpec((4 * cmid, C0), lambda n, j: (0, 0)),
            pl.BlockSpec((4 * cmid, 1), lambda n, j: (0, 0)),
            pl.BlockSpec((4 * cout1, cmid), lambda n, j: (0, 0)),
            pl.BlockSpec((4 * cout1, 1), lambda n, j: (0, 0)),
            pl.BlockSpec((4 * 4 * W, 4 * 4 * W), lambda n, j: (0, 0)),
        ],
        out_specs=pl.BlockSpec((1, cout1, 16, 4 * W), lambda n, j: (n, 0, j, 0)),
        compiler_params=pltpu.CompilerParams(
            dimension_semantics=("parallel", "parallel"),
            vmem_limit_bytes=_VMEM_LIMIT,
        ),
    )(x0, w1a_cat, b1a_cat, w1b_r, b1b_r, g1)

    x1 = f1.reshape(N, C0, HW)
    y2 = pl.pallas_call(
        _fpn2_body,
        out_shape=jax.ShapeDtypeStruct((N, cout2, nq, 4, 4 * W), jnp.float32),
        grid=(N, nq),
        in_specs=[
            pl.BlockSpec((1, C0, 4 * W), lambda n, j: (n, 0, j)),
            pl.BlockSpec((4 * cout2, C0), lambda n, j: (0, 0)),
            pl.BlockSpec((4 * cout2, 1), lambda n, j: (0, 0)),
            pl.BlockSpec((4 * 4 * W, 4 * 4 * W), lambda n, j: (0, 0)),
        ],
        out_specs=pl.BlockSpec((1, cout2, 1, 4, 4 * W), lambda n, j: (n, 0, j, 0, 0)),
        compiler_params=pltpu.CompilerParams(
            dimension_semantics=("parallel", "parallel"),
            vmem_limit_bytes=_VMEM_LIMIT,
        ),
    )(x1, w2_r, b2_r, g2)
    y2 = y2.reshape(N, cout2, 2 * H, 2 * W)

    Nc, C, Hp, Wp = f3.shape
    Hh, Wh = Hp // 2, Wp // 2
    R = Nc * C * Hh
    x3 = f3.reshape(R, 2 * Wp)
    tr = 1024 if R % 1024 == 0 else R
    i = np.arange(2 * Wp)[None, :, None]
    j = np.arange(Wh)[None, None, :]
    k = np.arange(4)[:, None, None]
    src = (k // 2) * Wp + 2 * j + (k % 2)
    sel = jnp.asarray((i == src).astype(np.float32))
    o4 = pl.pallas_call(
        _maxpool_body,
        out_shape=jax.ShapeDtypeStruct((R, Wh), f3.dtype),
        grid=(R // tr,),
        in_specs=[
            pl.BlockSpec((tr, 2 * Wp), lambda r: (r, 0)),
            pl.BlockSpec((4, 2 * Wp, Wh), lambda r: (0, 0, 0)),
        ],
        out_specs=pl.BlockSpec((tr, Wh), lambda r: (r, 0)),
        compiler_params=pltpu.CompilerParams(
            dimension_semantics=("parallel",),
            vmem_limit_bytes=_VMEM_LIMIT,
        ),
    )(x3, sel)
    y4 = o4.reshape(Nc, C, Hh, Wh)

    return [y1, y2, f2, y4]


# speedup vs baseline: 2.3990x; 1.5900x over previous
import math

import jax
import jax.numpy as jnp
from jax.experimental import pallas as pl
from jax.experimental.pallas import tpu as pltpu

_INV_SQRT2 = 1.0 / math.sqrt(2.0)
_VMEM_LIMIT = 64 * 1024 * 1024


def _fpn1_body(x_ref, w1a_ref, b1a_ref, w1b_ref, b1b_ref, o_ref):
    _, H_t, W, C0 = x_ref.shape
    cout = o_ref.shape[4] // 4
    cmid = w1b_ref.shape[0]
    T = H_t * W
    x = x_ref[0].reshape(T, C0).astype(jnp.bfloat16)
    z = jnp.dot(x, w1a_ref[...], preferred_element_type=jnp.float32) + b1a_ref[...]
    z = 0.5 * z * (1.0 + jax.lax.erf(z * _INV_SQRT2))
    zb = z.astype(jnp.bfloat16)
    for p in range(4):
        kh1, kw1 = p // 2, p % 2
        y = jnp.dot(zb[:, p * cmid:(p + 1) * cmid], w1b_ref[...],
                    preferred_element_type=jnp.float32) + b1b_ref[...]
        for kh2 in range(2):
            piece = y[:, kh2 * 2 * cout:(kh2 + 1) * 2 * cout].reshape(H_t, W, 2 * cout)
            o_ref[0, :, 2 * kh1 + kh2, :, 2 * kw1 * cout:2 * (kw1 + 1) * cout] = piece


def _fpn2_body(x_ref, w_ref, b_ref, o_ref):
    _, H_t, W, C0 = x_ref.shape
    co2 = o_ref.shape[4]
    x = x_ref[0].reshape(H_t * W, C0).astype(jnp.bfloat16)
    y = jnp.dot(x, w_ref[...], preferred_element_type=jnp.float32) + b_ref[...]
    for kh in range(2):
        piece = y[:, kh * co2:(kh + 1) * co2].reshape(H_t, W, co2)
        o_ref[0, :, kh, :, :] = piece


def _maxpool_body(x_ref, o_ref):
    a = jnp.maximum(x_ref[:, 0, :, 0, :], x_ref[:, 0, :, 1, :])
    b = jnp.maximum(x_ref[:, 1, :, 0, :], x_ref[:, 1, :, 1, :])
    o_ref[...] = jnp.maximum(a, b)


def kernel(f0, f1, f2, f3, k_w1a, k_b1a, k_w1b, k_b1b, k_w2, k_b2):
    N, C0, H, W = f0.shape
    cmid = k_w1a.shape[1]
    cout1 = k_w1b.shape[0] // 4
    cout2 = k_w2.shape[0] // 4
    bf = jnp.bfloat16

    w1aT = k_w1a.transpose(2, 0, 1).reshape(C0, 4 * cmid).astype(bf)
    b1a_row = k_b1a.reshape(1, 4 * cmid)
    w1bT = k_w1b.reshape(cout1, 4, cmid).transpose(2, 1, 0).reshape(cmid, 4 * cout1).astype(bf)
    b1b_row = k_b1b.reshape(cout1, 4).transpose(1, 0).reshape(1, 4 * cout1)
    w2T = k_w2.reshape(cout2, 4, C0).transpose(2, 1, 0).reshape(C0, 4 * cout2).astype(bf)
    b2_row = k_b2.reshape(cout2, 4).transpose(1, 0).reshape(1, 4 * cout2)

    H_t = 16
    nj = H // H_t

    x0 = f0.transpose(0, 2, 3, 1)
    o1 = pl.pallas_call(
        _fpn1_body,
        out_shape=jax.ShapeDtypeStruct((N, H, 4, W, 4 * cout1), jnp.float32),
        grid=(N, nj),
        in_specs=[
            pl.BlockSpec((1, H_t, W, C0), lambda n, j: (n, j, 0, 0)),
            pl.BlockSpec((C0, 4 * cmid), lambda n, j: (0, 0)),
            pl.BlockSpec((1, 4 * cmid), lambda n, j: (0, 0)),
            pl.BlockSpec((cmid, 4 * cout1), lambda n, j: (0, 0)),
            pl.BlockSpec((1, 4 * cout1), lambda n, j: (0, 0)),
        ],
        out_specs=pl.BlockSpec((1, H_t, 4, W, 4 * cout1), lambda n, j: (n, j, 0, 0, 0)),
        compiler_params=pltpu.CompilerParams(
            dimension_semantics=("parallel", "parallel"),
            vmem_limit_bytes=_VMEM_LIMIT,
        ),
    )(x0, w1aT, b1a_row, w1bT, b1b_row)
    y1 = o1.reshape(N, 4 * H, 4 * W, cout1).transpose(0, 3, 1, 2)

    x1 = f1.transpose(0, 2, 3, 1)
    o2 = pl.pallas_call(
        _fpn2_body,
        out_shape=jax.ShapeDtypeStruct((N, H, 2, W, 2 * cout2), jnp.float32),
        grid=(N, nj),
        in_specs=[
            pl.BlockSpec((1, H_t, W, C0), lambda n, j: (n, j, 0, 0)),
            pl.BlockSpec((C0, 4 * cout2), lambda n, j: (0, 0)),
            pl.BlockSpec((1, 4 * cout2), lambda n, j: (0, 0)),
        ],
        out_specs=pl.BlockSpec((1, H_t, 2, W, 2 * cout2), lambda n, j: (n, j, 0, 0, 0)),
        compiler_params=pltpu.CompilerParams(
            dimension_semantics=("parallel", "parallel"),
            vmem_limit_bytes=_VMEM_LIMIT,
        ),
    )(x1, w2T, b2_row)
    y2 = o2.reshape(N, 2 * H, 2 * W, cout2).transpose(0, 3, 1, 2)

    Hh, Wh = H // 2, W // 2
    x3 = f3.transpose(0, 2, 3, 1).reshape(N * Hh, 2, Wh, 2, C0)
    bh = 64 if (N * Hh) % 64 == 0 else N * Hh
    o4 = pl.pallas_call(
        _maxpool_body,
        out_shape=jax.ShapeDtypeStruct((N * Hh, Wh, C0), jnp.float32),
        grid=((N * Hh) // bh,),
        in_specs=[pl.BlockSpec((bh, 2, Wh, 2, C0), lambda r: (r, 0, 0, 0, 0))],
        out_specs=pl.BlockSpec((bh, Wh, C0), lambda r: (r, 0, 0)),
        compiler_params=pltpu.CompilerParams(
            dimension_semantics=("parallel",),
            vmem_limit_bytes=_VMEM_LIMIT,
        ),
    )(x3)
    y4 = o4.reshape(N, Hh, Wh, C0).transpose(0, 3, 1, 2)

    return [y1, y2, f2, y4]


# speedup vs baseline: 3.8593x; 1.6088x over previous
import math

import jax
import jax.numpy as jnp
from jax.experimental import pallas as pl
from jax.experimental.pallas import tpu as pltpu

_INV_SQRT2 = 1.0 / math.sqrt(2.0)
_VMEM_LIMIT = 64 * 1024 * 1024


def _fpn1_body(x_ref, w1a_ref, b1a_ref, w1b_ref, b1b_ref, o_ref):
    _, H_t, W, C0 = x_ref.shape
    cout = o_ref.shape[3]
    cmid = w1b_ref.shape[0]
    T = H_t * W
    x = x_ref[0].reshape(T, C0).astype(jnp.bfloat16)
    z = jnp.dot(x, w1a_ref[...], preferred_element_type=jnp.float32) + b1a_ref[...]
    z = 0.5 * z * (1.0 + jax.lax.erf(z * _INV_SQRT2))
    zb = z.astype(jnp.bfloat16)
    for p in range(4):
        kh1, kw1 = p // 2, p % 2
        y = jnp.dot(zb[:, p * cmid:(p + 1) * cmid], w1b_ref[...],
                    preferred_element_type=jnp.float32) + b1b_ref[...]
        for r in range(4):
            kh2, kw2 = r // 2, r % 2
            piece = y[:, r * cout:(r + 1) * cout].reshape(H_t, W, cout)
            o_ref[0, 2 * kh1 + kh2::4, 2 * kw1 + kw2::4, :] = piece


def _fpn2_body(x_ref, w_ref, b_ref, o_ref):
    _, H_t, W, C0 = x_ref.shape
    cout = o_ref.shape[3]
    x = x_ref[0].reshape(H_t * W, C0).astype(jnp.bfloat16)
    y = jnp.dot(x, w_ref[...], preferred_element_type=jnp.float32) + b_ref[...]
    for k in range(4):
        piece = y[:, k * cout:(k + 1) * cout].reshape(H_t, W, cout)
        o_ref[0, k // 2::2, k % 2::2, :] = piece


def _maxpool_body(x_ref, o_ref):
    a = jnp.maximum(x_ref[:, 0, :, 0, :], x_ref[:, 0, :, 1, :])
    b = jnp.maximum(x_ref[:, 1, :, 0, :], x_ref[:, 1, :, 1, :])
    o_ref[...] = jnp.maximum(a, b)


def kernel(f0, f1, f2, f3, k_w1a, k_b1a, k_w1b, k_b1b, k_w2, k_b2):
    N, C0, H, W = f0.shape
    cmid = k_w1a.shape[1]
    cout1 = k_w1b.shape[0] // 4
    cout2 = k_w2.shape[0] // 4
    bf = jnp.bfloat16

    w1aT = k_w1a.transpose(2, 0, 1).reshape(C0, 4 * cmid).astype(bf)
    b1a_row = k_b1a.reshape(1, 4 * cmid)
    w1bT = k_w1b.reshape(cout1, 4, cmid).transpose(2, 1, 0).reshape(cmid, 4 * cout1).astype(bf)
    b1b_row = k_b1b.reshape(cout1, 4).transpose(1, 0).reshape(1, 4 * cout1)
    w2T = k_w2.reshape(cout2, 4, C0).transpose(2, 1, 0).reshape(C0, 4 * cout2).astype(bf)
    b2_row = k_b2.reshape(cout2, 4).transpose(1, 0).reshape(1, 4 * cout2)

    H_t = 16
    nj = H // H_t

    x0 = f0.transpose(0, 2, 3, 1)
    o1 = pl.pallas_call(
        _fpn1_body,
        out_shape=jax.ShapeDtypeStruct((N, 4 * H, 4 * W, cout1), jnp.float32),
        grid=(N, nj),
        in_specs=[
            pl.BlockSpec((1, H_t, W, C0), lambda n, j: (n, j, 0, 0)),
            pl.BlockSpec((C0, 4 * cmid), lambda n, j: (0, 0)),
            pl.BlockSpec((1, 4 * cmid), lambda n, j: (0, 0)),
            pl.BlockSpec((cmid, 4 * cout1), lambda n, j: (0, 0)),
            pl.BlockSpec((1, 4 * cout1), lambda n, j: (0, 0)),
        ],
        out_specs=pl.BlockSpec((1, 4 * H_t, 4 * W, cout1), lambda n, j: (n, j, 0, 0)),
        compiler_params=pltpu.CompilerParams(
            dimension_semantics=("parallel", "parallel"),
            vmem_limit_bytes=_VMEM_LIMIT,
        ),
    )(x0, w1aT, b1a_row, w1bT, b1b_row)
    y1 = o1.transpose(0, 3, 1, 2)

    x1 = f1.transpose(0, 2, 3, 1)
    o2 = pl.pallas_call(
        _fpn2_body,
        out_shape=jax.ShapeDtypeStruct((N, 2 * H, 2 * W, cout2), jnp.float32),
        grid=(N, nj),
        in_specs=[
            pl.BlockSpec((1, H_t, W, C0), lambda n, j: (n, j, 0, 0)),
            pl.BlockSpec((C0, 4 * cout2), lambda n, j: (0, 0)),
            pl.BlockSpec((1, 4 * cout2), lambda n, j: (0, 0)),
        ],
        out_specs=pl.BlockSpec((1, 2 * H_t, 2 * W, cout2), lambda n, j: (n, j, 0, 0)),
        compiler_params=pltpu.CompilerParams(
            dimension_semantics=("parallel", "parallel"),
            vmem_limit_bytes=_VMEM_LIMIT,
        ),
    )(x1, w2T, b2_row)
    y2 = o2.transpose(0, 3, 1, 2)

    Hh, Wh = H // 2, W // 2
    x3 = f3.transpose(0, 2, 3, 1).reshape(N * Hh, 2, Wh, 2, C0)
    bh = 64 if (N * Hh) % 64 == 0 else N * Hh
    o4 = pl.pallas_call(
        _maxpool_body,
        out_shape=jax.ShapeDtypeStruct((N * Hh, Wh, C0), jnp.float32),
        grid=((N * Hh) // bh,),
        in_specs=[pl.BlockSpec((bh, 2, Wh, 2, C0), lambda r: (r, 0, 0, 0, 0))],
        out_specs=pl.BlockSpec((bh, Wh, C0), lambda r: (r, 0, 0)),
        compiler_params=pltpu.CompilerParams(
            dimension_semantics=("parallel",),
            vmem_limit_bytes=_VMEM_LIMIT,
        ),
    )(x3)
    y4 = o4.reshape(N, Hh, Wh, C0).transpose(0, 3, 1, 2)

    return [y1, y2, f2, y4]


# speedup vs baseline: 4.2188x; 1.0931x over previous
import math

import jax
import jax.numpy as jnp
from jax.experimental import pallas as pl
from jax.experimental.pallas import tpu as pltpu

_INV_SQRT2 = 1.0 / math.sqrt(2.0)
_VMEM_LIMIT = 64 * 1024 * 1024


def _fpn1_body(x_ref, w1a_ref, b1a_ref, w1b_ref, b1b_ref, o_ref):
    _, H_t, W, C0 = x_ref.shape
    cout = o_ref.shape[3]
    cmid = w1b_ref.shape[0]
    T = H_t * W
    x = x_ref[0].reshape(T, C0).astype(jnp.bfloat16)
    z = jnp.dot(x, w1a_ref[...], preferred_element_type=jnp.float32) + b1a_ref[...]
    z = 0.5 * z * (1.0 + jax.lax.erf(z * _INV_SQRT2))
    zb = z.astype(jnp.bfloat16)
    for p in range(4):
        kh1, kw1 = p // 2, p % 2
        y = jnp.dot(zb[:, p * cmid:(p + 1) * cmid], w1b_ref[...],
                    preferred_element_type=jnp.float32) + b1b_ref[...]
        for r in range(4):
            kh2, kw2 = r // 2, r % 2
            piece = y[:, r * cout:(r + 1) * cout].reshape(H_t, W, cout)
            o_ref[0, 2 * kh1 + kh2::4, 2 * kw1 + kw2::4, :] = piece


def _fpn2_body(x_ref, w_ref, b_ref, o_ref):
    _, H_t, W, C0 = x_ref.shape
    cout = o_ref.shape[3]
    x = x_ref[0].reshape(H_t * W, C0).astype(jnp.bfloat16)
    y = jnp.dot(x, w_ref[...], preferred_element_type=jnp.float32) + b_ref[...]
    for k in range(4):
        piece = y[:, k * cout:(k + 1) * cout].reshape(H_t, W, cout)
        o_ref[0, k // 2::2, k % 2::2, :] = piece


def _maxpool_body(x_ref, o_ref):
    bh, _, W, C = x_ref.shape
    m = jnp.maximum(x_ref[:, 0, :, :], x_ref[:, 1, :, :])
    m = jnp.max(m.reshape(bh * W // 2, 2, C), axis=1)
    o_ref[...] = m.reshape(bh, W // 2, C)


def _copy_body(x_ref, o_ref):
    o_ref[...] = x_ref[...]


def kernel(f0, f1, f2, f3, k_w1a, k_b1a, k_w1b, k_b1b, k_w2, k_b2):
    N, C0, H, W = f0.shape
    cmid = k_w1a.shape[1]
    cout1 = k_w1b.shape[0] // 4
    cout2 = k_w2.shape[0] // 4
    bf = jnp.bfloat16

    w1aT = k_w1a.transpose(2, 0, 1).reshape(C0, 4 * cmid).astype(bf)
    b1a_row = k_b1a.reshape(1, 4 * cmid)
    w1bT = k_w1b.reshape(cout1, 4, cmid).transpose(2, 1, 0).reshape(cmid, 4 * cout1).astype(bf)
    b1b_row = k_b1b.reshape(cout1, 4).transpose(1, 0).reshape(1, 4 * cout1)
    w2T = k_w2.reshape(cout2, 4, C0).transpose(2, 1, 0).reshape(C0, 4 * cout2).astype(bf)
    b2_row = k_b2.reshape(cout2, 4).transpose(1, 0).reshape(1, 4 * cout2)

    H_t = 16
    nj = H // H_t

    x0 = f0.transpose(0, 2, 3, 1)
    o1 = pl.pallas_call(
        _fpn1_body,
        out_shape=jax.ShapeDtypeStruct((N, 4 * H, 4 * W, cout1), jnp.float32),
        grid=(N, nj),
        in_specs=[
            pl.BlockSpec((1, H_t, W, C0), lambda n, j: (n, j, 0, 0)),
            pl.BlockSpec((C0, 4 * cmid), lambda n, j: (0, 0)),
            pl.BlockSpec((1, 4 * cmid), lambda n, j: (0, 0)),
            pl.BlockSpec((cmid, 4 * cout1), lambda n, j: (0, 0)),
            pl.BlockSpec((1, 4 * cout1), lambda n, j: (0, 0)),
        ],
        out_specs=pl.BlockSpec((1, 4 * H_t, 4 * W, cout1), lambda n, j: (n, j, 0, 0)),
        compiler_params=pltpu.CompilerParams(
            dimension_semantics=("parallel", "parallel"),
            vmem_limit_bytes=_VMEM_LIMIT,
        ),
    )(x0, w1aT, b1a_row, w1bT, b1b_row)
    y1 = o1.transpose(0, 3, 1, 2)

    x1 = f1.transpose(0, 2, 3, 1)
    o2 = pl.pallas_call(
        _fpn2_body,
        out_shape=jax.ShapeDtypeStruct((N, 2 * H, 2 * W, cout2), jnp.float32),
        grid=(N, nj),
        in_specs=[
            pl.BlockSpec((1, H_t, W, C0), lambda n, j: (n, j, 0, 0)),
            pl.BlockSpec((C0, 4 * cout2), lambda n, j: (0, 0)),
            pl.BlockSpec((1, 4 * cout2), lambda n, j: (0, 0)),
        ],
        out_specs=pl.BlockSpec((1, 2 * H_t, 2 * W, cout2), lambda n, j: (n, j, 0, 0)),
        compiler_params=pltpu.CompilerParams(
            dimension_semantics=("parallel", "parallel"),
            vmem_limit_bytes=_VMEM_LIMIT,
        ),
    )(x1, w2T, b2_row)
    y2 = o2.transpose(0, 3, 1, 2)

    Hh, Wh = H // 2, W // 2
    x3 = f3.transpose(0, 2, 3, 1).reshape(N * Hh, 2, W, C0)
    bh = 64 if (N * Hh) % 64 == 0 else N * Hh
    o4 = pl.pallas_call(
        _maxpool_body,
        out_shape=jax.ShapeDtypeStruct((N * Hh, Wh, C0), jnp.float32),
        grid=((N * Hh) // bh,),
        in_specs=[pl.BlockSpec((bh, 2, W, C0), lambda r: (r, 0, 0, 0))],
        out_specs=pl.BlockSpec((bh, Wh, C0), lambda r: (r, 0, 0)),
        compiler_params=pltpu.CompilerParams(
            dimension_semantics=("parallel",),
            vmem_limit_bytes=_VMEM_LIMIT,
        ),
    )(x3)
    y4 = o4.reshape(N, Hh, Wh, C0).transpose(0, 3, 1, 2)

    x2 = f2.transpose(0, 2, 3, 1).reshape(N * H * W, C0)
    bc = 2048 if (N * H * W) % 2048 == 0 else N * H * W
    o3 = pl.pallas_call(
        _copy_body,
        out_shape=jax.ShapeDtypeStruct((N * H * W, C0), jnp.float32),
        grid=((N * H * W) // bc,),
        in_specs=[pl.BlockSpec((bc, C0), lambda r: (r, 0))],
        out_specs=pl.BlockSpec((bc, C0), lambda r: (r, 0)),
        compiler_params=pltpu.CompilerParams(
            dimension_semantics=("parallel",),
            vmem_limit_bytes=_VMEM_LIMIT,
        ),
    )(x2)
    y3 = o3.reshape(N, H, W, C0).transpose(0, 3, 1, 2)

    return [y1, y2, y3, y4]
